# Initial kernel scaffold; baseline (speedup 1.0000x reference)
import jax
import jax.numpy as jnp
from jax.experimental import pallas as pl


def kernel(img, flow, depth):
    raise NotImplementedError("write your pallas kernel here")


# speedup vs baseline: 1.2475x; 1.2475x over previous
import functools

import jax
import jax.numpy as jnp
from jax import lax
from jax.experimental import pallas as pl
from jax.experimental.pallas import tpu as pltpu
from jax.experimental.pallas import tpu_sc as plsc

H = 512
W = 512
C = 3
N = H * W
NC = 2
NS = 16
NW = NC * NS
L = 16
SRC_PER_W = N // NW
ROWS_PER_W = H // NW
TGT_PER_W = ROWS_PER_W * W
CHUNK = 8192
NCHUNK = N // CHUNK
VPI = CHUNK // L
GCHUNK = 128
NGC = TGT_PER_W // GCHUNK
BIG = 1e30

@functools.lru_cache(maxsize=None)
def _mesh():
    return plsc.VectorSubcoreMesh(
        core_axis_name="c", subcore_axis_name="s",
        num_cores=NC, num_subcores=NS)


def _worker_id():
    return lax.axis_index("s") * NC + lax.axis_index("c")


def _round_nearest_even(v):
    vh = jnp.clip(v + 0.5, -1.0e6, 1.0e6)
    t = vh.astype(jnp.int32)
    r = jnp.where(vh < t.astype(jnp.float32), t - 1, t)
    tie = vh == r.astype(jnp.float32)
    odd = lax.bitwise_and(r, 1) == 1
    return jnp.where(tie & odd, r - 1, r)


def _splat_body(flow_hbm, depth_hbm, idx_hbm, d_hbm, fbuf, dbuf, ibuf, obuf):
    wid = _worker_id()
    base = wid * SRC_PER_W
    pltpu.sync_copy(flow_hbm.at[pl.ds(base * 2, SRC_PER_W * 2)], fbuf)
    pltpu.sync_copy(depth_hbm.at[pl.ds(base, SRC_PER_W)], dbuf)
    lanes = lax.iota(jnp.int32, L)

    def body(j, carry):
        p = base + j * L + lanes
        y = lax.shift_right_logical(p, 9)
        x = lax.bitwise_and(p, W - 1)
        g = j * (2 * L) + lanes * 2
        fx = plsc.load_gather(fbuf, [g])
        fy = plsc.load_gather(fbuf, [g + 1])
        tx = _round_nearest_even(x.astype(jnp.float32) + fx)
        ty = _round_nearest_even(y.astype(jnp.float32) + fy)
        valid = (tx >= 0) & (tx < W) & (ty >= 0) & (ty < H)
        tidx = jnp.where(valid, ty * W + tx, N)
        dd = jnp.where(valid, dbuf[pl.ds(j * L, L)], BIG)
        ibuf[pl.ds(j * L, L)] = tidx
        obuf[pl.ds(j * L, L)] = dd
        return carry

    lax.fori_loop(0, SRC_PER_W // L, body, 0)
    pltpu.sync_copy(ibuf, idx_hbm.at[pl.ds(base, SRC_PER_W)])
    pltpu.sync_copy(obuf, d_hbm.at[pl.ds(base, SRC_PER_W)])


@functools.lru_cache(maxsize=None)
def _pass_a():
    return pl.kernel(
        _splat_body,
        mesh=_mesh(),
        out_type=[
            jax.ShapeDtypeStruct((N,), jnp.int32),
            jax.ShapeDtypeStruct((N,), jnp.float32),
        ],
        scratch_types=[
            pltpu.VMEM((2 * SRC_PER_W,), jnp.float32),
            pltpu.VMEM((SRC_PER_W,), jnp.float32),
            pltpu.VMEM((SRC_PER_W,), jnp.int32),
            pltpu.VMEM((SRC_PER_W,), jnp.float32),
        ],
        compiler_params=pltpu.CompilerParams(needs_layout_passes=False),
    )


def _zbuf_body(idx_hbm, d_hbm, img_hbm, out_hbm, zbuf, oslab, icbuf, dcbuf,
               gbuf):
    wid = _worker_id()
    tbase = wid * TGT_PER_W
    tend = tbase + TGT_PER_W
    lanes = lax.iota(jnp.int32, L)

    def initz(j, carry):
        zbuf[pl.ds(j * L, L)] = jnp.full((L,), BIG, jnp.float32)
        return carry

    lax.fori_loop(0, TGT_PER_W // L, initz, 0)

    def inito(j, carry):
        oslab[pl.ds(j * L, L)] = jnp.zeros((L,), jnp.float32)
        return carry

    lax.fori_loop(0, TGT_PER_W * C // L, inito, 0)

    def chunk1(c, carry):
        pltpu.sync_copy(idx_hbm.at[pl.ds(c * CHUNK, CHUNK)], icbuf)
        pltpu.sync_copy(d_hbm.at[pl.ds(c * CHUNK, CHUNK)], dcbuf)

        def it(j, cc):
            ti = icbuf[pl.ds(j * L, L)]
            m = (ti >= tbase) & (ti < tend)

            @pl.when(jnp.any(m))
            def _():
                dd = dcbuf[pl.ds(j * L, L)]
                loc = lax.bitwise_and(ti - tbase, TGT_PER_W - 1)
                cur = plsc.load_gather(zbuf, [loc], mask=m)
                win = m & (dd < cur)

                def wbody(w):
                    plsc.store_scatter(zbuf, [loc], dd, mask=w)
                    cur2 = plsc.load_gather(zbuf, [loc], mask=w)
                    return w & (dd < cur2)

                lax.while_loop(lambda w: jnp.any(w), wbody, win)

            return cc

        lax.fori_loop(0, VPI, it, 0)
        return carry

    lax.fori_loop(0, NCHUNK, chunk1, 0)

    def chunk2(c, carry):
        pltpu.sync_copy(idx_hbm.at[pl.ds(c * CHUNK, CHUNK)], icbuf)
        pltpu.sync_copy(d_hbm.at[pl.ds(c * CHUNK, CHUNK)], dcbuf)
        pltpu.sync_copy(img_hbm.at[pl.ds(c * CHUNK * C, CHUNK * C)], gbuf)

        def it(j, cc):
            ti = icbuf[pl.ds(j * L, L)]
            m = (ti >= tbase) & (ti < tend)

            @pl.when(jnp.any(m))
            def _():
                dd = dcbuf[pl.ds(j * L, L)]
                loc = lax.bitwise_and(ti - tbase, TGT_PER_W - 1)
                cur = plsc.load_gather(zbuf, [loc], mask=m)
                winm = m & (dd == cur)

                @pl.when(jnp.any(winm))
                def _():
                    g = (j * L + lanes) * C
                    loc3 = loc * C
                    for ch in range(C):
                        v = plsc.load_gather(gbuf, [g + ch], mask=winm)
                        plsc.store_scatter(oslab, [loc3 + ch], v, mask=winm)

            return cc

        lax.fori_loop(0, VPI, it, 0)
        return carry

    lax.fori_loop(0, NCHUNK, chunk2, 0)

    pltpu.sync_copy(oslab, out_hbm.at[pl.ds(tbase * C, TGT_PER_W * C)])


@functools.lru_cache(maxsize=None)
def _pass_b():
    return pl.kernel(
        _zbuf_body,
        mesh=_mesh(),
        out_type=jax.ShapeDtypeStruct((N * C,), jnp.float32),
        scratch_types=[
            pltpu.VMEM((TGT_PER_W,), jnp.float32),
            pltpu.VMEM((TGT_PER_W * C,), jnp.float32),
            pltpu.VMEM((CHUNK,), jnp.int32),
            pltpu.VMEM((CHUNK,), jnp.float32),
            pltpu.VMEM((CHUNK * C,), jnp.float32),
        ],
        compiler_params=pltpu.CompilerParams(needs_layout_passes=False),
    )


def kernel(img, flow, depth):
    flow_flat = flow.reshape(-1)
    depth_flat = depth.reshape(-1)
    img_flat = img.reshape(-1)
    tidx, d = _pass_a()(flow_flat, depth_flat)
    out_flat = _pass_b()(tidx, d, img_flat)
    return out_flat.reshape(H, W, C)


# speedup vs baseline: 3.9838x; 3.1933x over previous
import functools

import jax
import jax.numpy as jnp
from jax import lax
from jax.experimental import pallas as pl
from jax.experimental.pallas import tpu as pltpu
from jax.experimental.pallas import tpu_sc as plsc

H = 512
W = 512
C = 3
N = H * W
NC = 2
NS = 16
NW = NC * NS
L = 16
CHUNK = N // NW
VPI = CHUNK // L
BANDSHIFT = 13
OVBLK = 1024
NOVBLK = CHUNK // OVBLK
BIG = 1e30


@functools.lru_cache(maxsize=None)
def _mesh():
    return plsc.VectorSubcoreMesh(
        core_axis_name="c", subcore_axis_name="s",
        num_cores=NC, num_subcores=NS)


def _worker_id():
    return lax.axis_index("s") * NC + lax.axis_index("c")


def _round_nearest_even(v):
    vh = jnp.clip(v + 0.5, -1.0e6, 1.0e6)
    t = vh.astype(jnp.int32)
    r = jnp.where(vh < t.astype(jnp.float32), t - 1, t)
    tie = vh == r.astype(jnp.float32)
    odd = lax.bitwise_and(r, 1) == 1
    return jnp.where(tie & odd, r - 1, r)


def _splat_body(flow_hbm, depth_hbm, img_hbm,
                idx_hbm, d_hbm, cnt_hbm, ovt_hbm, ovd_hbm, ovr_hbm, ovg_hbm,
                ovb_hbm,
                fbuf, dbuf, gbuf, ibuf, obuf, tvb, dvb, rvb, gvb, bvb):
    wid = _worker_id()
    base = wid * CHUNK
    pltpu.sync_copy(flow_hbm.at[pl.ds(base * 2, CHUNK * 2)], fbuf)
    pltpu.sync_copy(depth_hbm.at[pl.ds(base, CHUNK)], dbuf)
    pltpu.sync_copy(img_hbm.at[pl.ds(base * C, CHUNK * C)], gbuf)
    lanes = lax.iota(jnp.int32, L)

    def body(j, cur):
        p = base + j * L + lanes
        y = lax.shift_right_logical(p, 9)
        x = lax.bitwise_and(p, W - 1)
        g = j * (2 * L) + lanes * 2
        fx = plsc.load_gather(fbuf, [g])
        fy = plsc.load_gather(fbuf, [g + 1])
        tx = _round_nearest_even(x.astype(jnp.float32) + fx)
        ty = _round_nearest_even(y.astype(jnp.float32) + fy)
        valid = (tx >= 0) & (tx < W) & (ty >= 0) & (ty < H)
        tidx = jnp.where(valid, ty * W + tx, N)
        dd = jnp.where(valid, dbuf[pl.ds(j * L, L)], BIG)
        ibuf[pl.ds(j * L, L)] = tidx
        obuf[pl.ds(j * L, L)] = dd
        band = lax.shift_right_logical(tidx, BANDSHIFT)
        ovf = valid & ((band + 1 < wid) | (band > wid + 1))
        ncnt = jnp.sum(ovf.astype(jnp.int32))

        @pl.when(jnp.any(ovf))
        def _():
            pos = cur + plsc.cumsum(ovf.astype(jnp.int32)) - 1
            plsc.store_scatter(tvb, [pos], tidx, mask=ovf)
            plsc.store_scatter(dvb, [pos], dd, mask=ovf)
            gi = (j * L + lanes) * C
            for ch, vb in ((0, rvb), (1, gvb), (2, bvb)):
                v = plsc.load_gather(gbuf, [gi + ch], mask=ovf)
                plsc.store_scatter(vb, [pos], v, mask=ovf)

        return cur + ncnt

    cur = lax.fori_loop(0, VPI, body, jnp.int32(0))
    pltpu.sync_copy(ibuf, idx_hbm.at[pl.ds(base, CHUNK)])
    pltpu.sync_copy(obuf, d_hbm.at[pl.ds(base, CHUNK)])
    ibuf[pl.ds(0, L)] = jnp.zeros((L,), jnp.int32) + cur
    pltpu.sync_copy(ibuf.at[pl.ds(0, L)], cnt_hbm.at[pl.ds(wid * L, L)])

    def wr(b, carry):
        @pl.when(b * OVBLK < cur)
        def _():
            off = b * OVBLK
            for vb, hb in ((tvb, ovt_hbm), (dvb, ovd_hbm), (rvb, ovr_hbm),
                           (gvb, ovg_hbm), (bvb, ovb_hbm)):
                pltpu.sync_copy(vb.at[pl.ds(off, OVBLK)],
                                hb.at[pl.ds(base + off, OVBLK)])
        return carry

    lax.fori_loop(0, NOVBLK, wr, 0)


@functools.lru_cache(maxsize=None)
def _pass_a():
    return pl.kernel(
        _splat_body,
        mesh=_mesh(),
        out_type=[
            jax.ShapeDtypeStruct((N,), jnp.int32),
            jax.ShapeDtypeStruct((N,), jnp.float32),
            jax.ShapeDtypeStruct((NW * L,), jnp.int32),
            jax.ShapeDtypeStruct((N,), jnp.int32),
            jax.ShapeDtypeStruct((N,), jnp.float32),
            jax.ShapeDtypeStruct((N,), jnp.float32),
            jax.ShapeDtypeStruct((N,), jnp.float32),
            jax.ShapeDtypeStruct((N,), jnp.float32),
        ],
        scratch_types=[
            pltpu.VMEM((2 * CHUNK,), jnp.float32),
            pltpu.VMEM((CHUNK,), jnp.float32),
            pltpu.VMEM((CHUNK * C,), jnp.float32),
            pltpu.VMEM((CHUNK,), jnp.int32),
            pltpu.VMEM((CHUNK,), jnp.float32),
            pltpu.VMEM((CHUNK,), jnp.int32),
            pltpu.VMEM((CHUNK,), jnp.float32),
            pltpu.VMEM((CHUNK,), jnp.float32),
            pltpu.VMEM((CHUNK,), jnp.float32),
            pltpu.VMEM((CHUNK,), jnp.float32),
        ],
        compiler_params=pltpu.CompilerParams(needs_layout_passes=False),
    )


def _zbuf_body(idx_hbm, d_hbm, img_hbm, cnt_hbm, ovt_hbm, ovd_hbm, ovr_hbm,
               ovg_hbm, ovb_hbm, out_hbm,
               zbuf, oslab, cbuf, icbuf, dcbuf, gbuf):
    wid = _worker_id()
    tbase = wid * CHUNK
    tend = tbase + CHUNK
    lanes = lax.iota(jnp.int32, L)

    pltpu.sync_copy(cnt_hbm, cbuf)

    def initz(j, carry):
        zbuf[pl.ds(j * L, L)] = jnp.full((L,), BIG, jnp.float32)
        return carry

    lax.fori_loop(0, CHUNK // L, initz, 0)

    def inito(j, carry):
        oslab[pl.ds(j * L, L)] = jnp.zeros((L,), jnp.float32)
        return carry

    lax.fori_loop(0, CHUNK * C // L, inito, 0)

    def _zupdate(ti, dd, m):
        @pl.when(jnp.any(m))
        def _():
            loc = lax.bitwise_and(ti - tbase, CHUNK - 1)
            cur = plsc.load_gather(zbuf, [loc], mask=m)
            win = m & (dd < cur)

            def wbody(w):
                plsc.store_scatter(zbuf, [loc], dd, mask=w)
                cur2 = plsc.load_gather(zbuf, [loc], mask=w)
                return w & (dd < cur2)

            lax.while_loop(lambda w: jnp.any(w), wbody, win)

    def _vscatter(ti, dd, m, gidx, src_ref):
        @pl.when(jnp.any(m))
        def _():
            loc = lax.bitwise_and(ti - tbase, CHUNK - 1)
            cur = plsc.load_gather(zbuf, [loc], mask=m)
            winm = m & (dd == cur)

            @pl.when(jnp.any(winm))
            def _():
                loc3 = loc * C
                for ch in range(C):
                    v = plsc.load_gather(src_ref, [gidx + ch], mask=winm)
                    plsc.store_scatter(oslab, [loc3 + ch], v, mask=winm)

    def near1(off, carry):
        sb = wid + off - 1

        @pl.when((sb >= 0) & (sb < NW))
        def _():
            pltpu.sync_copy(idx_hbm.at[pl.ds(sb * CHUNK, CHUNK)], icbuf)
            pltpu.sync_copy(d_hbm.at[pl.ds(sb * CHUNK, CHUNK)], dcbuf)

            def it(j, cc):
                ti = icbuf[pl.ds(j * L, L)]
                m = (ti >= tbase) & (ti < tend)
                _zupdate(ti, dcbuf[pl.ds(j * L, L)], m)
                return cc

            lax.fori_loop(0, VPI, it, 0)

        return carry

    lax.fori_loop(0, 3, near1, 0)

    def _ovcnt(seg):
        return jnp.max(cbuf[pl.ds(seg * L, L)])

    def ov1(seg, carry):
        cnt = _ovcnt(seg)

        def blk(b, cc):
            @pl.when(b * OVBLK < cnt)
            def _():
                sbase = seg * CHUNK + b * OVBLK
                pltpu.sync_copy(ovt_hbm.at[pl.ds(sbase, OVBLK)],
                                icbuf.at[pl.ds(0, OVBLK)])
                pltpu.sync_copy(ovd_hbm.at[pl.ds(sbase, OVBLK)],
                                dcbuf.at[pl.ds(0, OVBLK)])

                def it(j, c2):
                    ti = icbuf[pl.ds(j * L, L)]
                    posm = (b * OVBLK + j * L + lanes) < cnt
                    m = posm & (ti >= tbase) & (ti < tend)
                    _zupdate(ti, dcbuf[pl.ds(j * L, L)], m)
                    return c2

                lax.fori_loop(0, OVBLK // L, it, 0)

            return cc

        lax.fori_loop(0, NOVBLK, blk, 0)
        return carry

    lax.fori_loop(0, NW, ov1, 0)

    def near2(off, carry):
        sb = wid + off - 1

        @pl.when((sb >= 0) & (sb < NW))
        def _():
            pltpu.sync_copy(idx_hbm.at[pl.ds(sb * CHUNK, CHUNK)], icbuf)
            pltpu.sync_copy(d_hbm.at[pl.ds(sb * CHUNK, CHUNK)], dcbuf)
            pltpu.sync_copy(img_hbm.at[pl.ds(sb * CHUNK * C, CHUNK * C)], gbuf)

            def it(j, cc):
                ti = icbuf[pl.ds(j * L, L)]
                m = (ti >= tbase) & (ti < tend)
                _vscatter(ti, dcbuf[pl.ds(j * L, L)], m,
                          (j * L + lanes) * C, gbuf)
                return cc

            lax.fori_loop(0, VPI, it, 0)

        return carry

    lax.fori_loop(0, 3, near2, 0)

    def ov2(seg, carry):
        cnt = _ovcnt(seg)

        def blk(b, cc):
            @pl.when(b * OVBLK < cnt)
            def _():
                sbase = seg * CHUNK + b * OVBLK
                pltpu.sync_copy(ovt_hbm.at[pl.ds(sbase, OVBLK)],
                                icbuf.at[pl.ds(0, OVBLK)])
                pltpu.sync_copy(ovd_hbm.at[pl.ds(sbase, OVBLK)],
                                dcbuf.at[pl.ds(0, OVBLK)])
                pltpu.sync_copy(ovr_hbm.at[pl.ds(sbase, OVBLK)],
                                gbuf.at[pl.ds(0, OVBLK)])
                pltpu.sync_copy(ovg_hbm.at[pl.ds(sbase, OVBLK)],
                                gbuf.at[pl.ds(OVBLK, OVBLK)])
                pltpu.sync_copy(ovb_hbm.at[pl.ds(sbase, OVBLK)],
                                gbuf.at[pl.ds(2 * OVBLK, OVBLK)])

                def it(j, c2):
                    ti = icbuf[pl.ds(j * L, L)]
                    dd = dcbuf[pl.ds(j * L, L)]
                    posm = (b * OVBLK + j * L + lanes) < cnt
                    m = posm & (ti >= tbase) & (ti < tend)

                    @pl.when(jnp.any(m))
                    def _():
                        loc = lax.bitwise_and(ti - tbase, CHUNK - 1)
                        cur = plsc.load_gather(zbuf, [loc], mask=m)
                        winm = m & (dd == cur)

                        @pl.when(jnp.any(winm))
                        def _():
                            loc3 = loc * C
                            e = j * L + lanes
                            for ch in range(C):
                                v = plsc.load_gather(
                                    gbuf, [ch * OVBLK + e], mask=winm)
                                plsc.store_scatter(
                                    oslab, [loc3 + ch], v, mask=winm)

                    return c2

                lax.fori_loop(0, OVBLK // L, it, 0)

            return cc

        lax.fori_loop(0, NOVBLK, blk, 0)
        return carry

    lax.fori_loop(0, NW, ov2, 0)

    pltpu.sync_copy(oslab, out_hbm.at[pl.ds(tbase * C, CHUNK * C)])


@functools.lru_cache(maxsize=None)
def _pass_b():
    return pl.kernel(
        _zbuf_body,
        mesh=_mesh(),
        out_type=jax.ShapeDtypeStruct((N * C,), jnp.float32),
        scratch_types=[
            pltpu.VMEM((CHUNK,), jnp.float32),
            pltpu.VMEM((CHUNK * C,), jnp.float32),
            pltpu.VMEM((NW * L,), jnp.int32),
            pltpu.VMEM((CHUNK,), jnp.int32),
            pltpu.VMEM((CHUNK,), jnp.float32),
            pltpu.VMEM((CHUNK * C,), jnp.float32),
        ],
        compiler_params=pltpu.CompilerParams(needs_layout_passes=False),
    )


def kernel(img, flow, depth):
    flow_flat = flow.reshape(-1)
    depth_flat = depth.reshape(-1)
    img_flat = img.reshape(-1)
    tidx, d, cnt, ovt, ovd, ovr, ovg, ovb = _pass_a()(
        flow_flat, depth_flat, img_flat)
    out_flat = _pass_b()(tidx, d, img_flat, cnt, ovt, ovd, ovr, ovg, ovb)
    return out_flat.reshape(H, W, C)
